# baseline (speedup 1.0000x reference)
import functools

import jax
import jax.numpy as jnp
from jax import lax
from jax.experimental import pallas as pl
from jax.experimental.pallas import tpu as pltpu
from jax.experimental.pallas import tpu_sc as plsc

BATCH = 16384
EMBED_DIM = 32
NUM_ROWS = 1_000_000
CROWS = 999_936
TAIL = NUM_ROWS - CROWS
NUM_CORES = 2
NUM_SUBCORES = 16
LANES = 16
NUM_WORKERS = NUM_CORES * NUM_SUBCORES
BW = BATCH // NUM_WORKERS
GROUPS = BW // LANES
NW_IDX = BW * EMBED_DIM
IDX_CHUNK = 128
N_CHUNKS = NW_IDX // IDX_CHUNK
B_CHUNKS = BW // IDX_CHUNK
DRAIN_LAG = 32
FLAT_C = EMBED_DIM * CROWS
TAIL_N = EMBED_DIM * TAIL

_mesh = plsc.VectorSubcoreMesh(core_axis_name="c", subcore_axis_name="s")


def _flatten_pair_body(u_hbm, i_hbm, uo_hbm, io_hbm, sem):
    copies = []
    for src, dst in ((u_hbm, uo_hbm), (i_hbm, io_hbm)):
        for r in range(EMBED_DIM):
            copies.append(pltpu.make_async_copy(
                src.at[r, pl.ds(0, CROWS)], dst.at[pl.ds(r * CROWS, CROWS)],
                sem))
    for cp in copies:
        cp.start()
    for cp in copies:
        cp.wait()


_flatten_pair = pl.pallas_call(
    _flatten_pair_body,
    in_specs=[pl.BlockSpec(memory_space=pltpu.MemorySpace.HBM)] * 2,
    out_specs=[pl.BlockSpec(memory_space=pltpu.MemorySpace.HBM)] * 2,
    out_shape=[jax.ShapeDtypeStruct((FLAT_C,), jnp.float32)] * 2,
    scratch_shapes=[pltpu.SemaphoreType.DMA],
)


@functools.partial(
    pl.kernel,
    mesh=_mesh,
    out_type=jax.ShapeDtypeStruct((BATCH,), jnp.float32),
    compiler_params=pltpu.CompilerParams(
        needs_layout_passes=False, use_tc_tiling_on_sc=False),
    scratch_types=[
        pltpu.VMEM((BW,), jnp.int32),
        pltpu.VMEM((BW,), jnp.int32),
        pltpu.VMEM((NW_IDX,), jnp.int32),
        pltpu.VMEM((NW_IDX,), jnp.int32),
        pltpu.VMEM((NW_IDX,), jnp.float32),
        pltpu.VMEM((NW_IDX,), jnp.float32),
        pltpu.VMEM((BW,), jnp.float32),
        pltpu.VMEM((BW,), jnp.float32),
        pltpu.VMEM((TAIL_N,), jnp.float32),
        pltpu.VMEM((TAIL_N,), jnp.float32),
        pltpu.VMEM((BW,), jnp.float32),
        pltpu.SemaphoreType.DMA,
        pltpu.SemaphoreType.DMA,
    ],
)
def _mf_sc_kernel(uid_hbm, iid_hbm, uef_hbm, ub_hbm, ief_hbm, ib_hbm,
                  ut_hbm, it_hbm, out_hbm,
                  uid_v, iid_v, idx_u, idx_i, pu_v, qi_v, pb_v, qb_v,
                  ut_v, it_v, out_v, sem, bsem):
    wid = lax.axis_index("s") * NUM_CORES + lax.axis_index("c")
    base = wid * BW

    pltpu.sync_copy(uid_hbm.at[pl.ds(base, BW)], uid_v)
    pltpu.sync_copy(iid_hbm.at[pl.ds(base, BW)], iid_v)
    pltpu.sync_copy(ut_hbm, ut_v)
    pltpu.sync_copy(it_hbm, it_v)

    bias_copies = []
    for c in range(B_CHUNKS):
        s = pl.ds(c * IDX_CHUNK, IDX_CHUNK)
        bias_copies.append(pltpu.async_copy(ub_hbm.at[uid_v.at[s]], pb_v.at[s], bsem))
        bias_copies.append(pltpu.async_copy(ib_hbm.at[iid_v.at[s]], qb_v.at[s], bsem))

    def gen_body(g, carry):
        for ids_ref, idx_ref in ((uid_v, idx_u), (iid_v, idx_i)):
            idv = ids_ref[pl.ds(g * LANES, LANES)]
            idc = jnp.minimum(idv, CROWS - 1)
            for d in range(EMBED_DIM):
                idx_ref[pl.ds(d * BW + g * LANES, LANES)] = idc + d * CROWS
        return carry

    lax.fori_loop(0, GROUPS, gen_body, 0, unroll=False)

    def drain_pair():
        s0 = pl.ds(0, IDX_CHUNK)
        pltpu.make_async_copy(uef_hbm.at[idx_u.at[s0]], pu_v.at[s0], sem).wait()
        pltpu.make_async_copy(ief_hbm.at[idx_i.at[s0]], qi_v.at[s0], sem).wait()

    def dma_body(c, carry):
        s = pl.ds(c * IDX_CHUNK, IDX_CHUNK)
        pltpu.async_copy(uef_hbm.at[idx_u.at[s]], pu_v.at[s], sem)
        pltpu.async_copy(ief_hbm.at[idx_i.at[s]], qi_v.at[s], sem)

        @pl.when(c >= DRAIN_LAG)
        def _():
            drain_pair()

        return carry

    lax.fori_loop(0, N_CHUNKS, dma_body, 0, unroll=False)
    for _ in range(DRAIN_LAG):
        drain_pair()
    for cp in bias_copies:
        cp.wait()

    def compute_body(g, carry):
        s = pl.ds(g * LANES, LANES)
        uidv = uid_v[s]
        iidv = iid_v[s]
        u_tail = uidv >= CROWS
        i_tail = iidv >= CROWS
        ut_off = jnp.maximum(uidv - CROWS, 0)
        it_off = jnp.maximum(iidv - CROWS, 0)
        acc = pb_v[s] + qb_v[s]
        for d in range(EMBED_DIM):
            sd = pl.ds(d * BW + g * LANES, LANES)
            uval = jnp.where(u_tail,
                             plsc.load_gather(ut_v, [ut_off + d * TAIL]),
                             pu_v[sd])
            ival = jnp.where(i_tail,
                             plsc.load_gather(it_v, [it_off + d * TAIL]),
                             qi_v[sd])
            acc = acc + uval * ival
        out_v[s] = acc
        return carry

    lax.fori_loop(0, GROUPS, compute_body, 0, unroll=False)

    pltpu.sync_copy(out_v, out_hbm.at[pl.ds(base, BW)])


def kernel(user_id, item_id, user_embedding, user_bias, item_embedding, item_bias):
    uid = user_id.astype(jnp.int32)
    iid = item_id.astype(jnp.int32)
    uet = user_embedding.T
    iet = item_embedding.T
    uef, ief = _flatten_pair(uet, iet)
    u_tail = uet[:, CROWS:].reshape(-1)
    i_tail = iet[:, CROWS:].reshape(-1)
    return _mf_sc_kernel(uid, iid, uef, user_bias.reshape(-1),
                         ief, item_bias.reshape(-1), u_tail, i_tail)


# speedup vs baseline: 8.7051x; 8.7051x over previous
import functools

import jax
import jax.numpy as jnp
from jax import lax
from jax.experimental import pallas as pl
from jax.experimental.pallas import tpu as pltpu
from jax.experimental.pallas import tpu_sc as plsc

BATCH = 16384
EMBED_DIM = 32
NUM_CORES = 2
NUM_SUBCORES = 16
LANES = 16
NUM_WORKERS = NUM_CORES * NUM_SUBCORES
BW = BATCH // NUM_WORKERS
IDX_CHUNK = 128
N_CHUNKS = BW // IDX_CHUNK
GROUPS = BW // LANES

_mesh = plsc.VectorSubcoreMesh(core_axis_name="c", subcore_axis_name="s")


@functools.partial(
    pl.kernel,
    mesh=_mesh,
    out_type=jax.ShapeDtypeStruct((BATCH,), jnp.float32),
    compiler_params=pltpu.CompilerParams(
        needs_layout_passes=False, use_tc_tiling_on_sc=False),
    scratch_types=[
        pltpu.VMEM((BW,), jnp.int32),
        pltpu.VMEM((BW,), jnp.int32),
        pltpu.VMEM((BW, EMBED_DIM), jnp.float32),
        pltpu.VMEM((BW, EMBED_DIM), jnp.float32),
        pltpu.VMEM((BW,), jnp.float32),
        pltpu.VMEM((BW,), jnp.float32),
        pltpu.VMEM((BW,), jnp.float32),
        pltpu.SemaphoreType.DMA,
    ],
)
def _mf_sc_kernel(uid_hbm, iid_hbm, ue_hbm, ub_hbm, ie_hbm, ib_hbm, out_hbm,
                  uid_v, iid_v, p_v, q_v, pb_v, qb_v, out_v, sem):
    wid = lax.axis_index("s") * NUM_CORES + lax.axis_index("c")
    base = wid * BW

    pltpu.sync_copy(uid_hbm.at[pl.ds(base, BW)], uid_v)
    pltpu.sync_copy(iid_hbm.at[pl.ds(base, BW)], iid_v)

    copies = []
    for c in range(N_CHUNKS):
        s = pl.ds(c * IDX_CHUNK, IDX_CHUNK)
        copies.append(pltpu.async_copy(ue_hbm.at[uid_v.at[s]], p_v.at[s], sem))
        copies.append(pltpu.async_copy(ie_hbm.at[iid_v.at[s]], q_v.at[s], sem))
        copies.append(pltpu.async_copy(ub_hbm.at[uid_v.at[s]], pb_v.at[s], sem))
        copies.append(pltpu.async_copy(ib_hbm.at[iid_v.at[s]], qb_v.at[s], sem))
    for cp in copies:
        cp.wait()

    lanes = lax.iota(jnp.int32, LANES)

    def group_body(g, carry):
        rows = g * LANES + lanes
        acc = plsc.load_gather(pb_v, [rows])
        acc = acc + plsc.load_gather(qb_v, [rows])
        for d in range(EMBED_DIM):
            dcol = jnp.full((LANES,), d, jnp.int32)
            acc = acc + (plsc.load_gather(p_v, [rows, dcol])
                         * plsc.load_gather(q_v, [rows, dcol]))
        plsc.store_scatter(out_v, [rows], acc)
        return carry

    lax.fori_loop(0, GROUPS, group_body, 0)

    pltpu.sync_copy(out_v, out_hbm.at[pl.ds(base, BW)])


def kernel(user_id, item_id, user_embedding, user_bias, item_embedding, item_bias):
    uid = user_id.astype(jnp.int32)
    iid = item_id.astype(jnp.int32)
    return _mf_sc_kernel(uid, iid, user_embedding, user_bias.reshape(-1),
                         item_embedding, item_bias.reshape(-1))
